# Initial kernel scaffold; baseline (speedup 1.0000x reference)
import jax
import jax.numpy as jnp
from jax.experimental import pallas as pl


def kernel(partial, outs, out1_feat, mean_mst_dis):
    raise NotImplementedError("write your pallas kernel here")


# speedup vs baseline: 20.5015x; 20.5015x over previous
import functools

import jax
import jax.numpy as jnp
from jax import lax
from jax.experimental import pallas as pl
from jax.experimental.pallas import tpu as pltpu
from jax.experimental.pallas import tpu_sc as plsc


def _mds_body(x_ref, y_ref, z_ref, s2_ref, out_ref):
    x = x_ref[...]
    y = y_ref[...]
    z = z_ref[...]
    s2 = s2_ref[...]
    B, n = x.shape
    npoint = out_ref.shape[1]
    lane_n = lax.broadcasted_iota(jnp.int32, (B, n), 1)
    lane_np = lax.broadcasted_iota(jnp.int32, (B, npoint), 1)

    def body(j, state):
        density, px, py, pz, acc = state
        dx = x - px
        dy = y - py
        dz = z - pz
        d2 = (dx * dx + dz * dz) + dy * dy
        density = density + jnp.exp(-d2 / s2)
        m = jnp.min(density, axis=1, keepdims=True)
        idxv = jnp.min(jnp.where(density == m, lane_n, n), axis=1, keepdims=True)
        acc = jnp.where(lane_np == j, idxv, acc)
        sel = lane_n == idxv
        px = jnp.sum(jnp.where(sel, x, 0.0), axis=1, keepdims=True)
        py = jnp.sum(jnp.where(sel, y, 0.0), axis=1, keepdims=True)
        pz = jnp.sum(jnp.where(sel, z, 0.0), axis=1, keepdims=True)
        return density, px, py, pz, acc

    density0 = jnp.zeros((B, n), x.dtype)
    acc0 = jnp.zeros((B, npoint), jnp.int32)
    state0 = (density0, x[:, 0:1], y[:, 0:1], z[:, 0:1], acc0)
    _, _, _, _, acc = lax.fori_loop(1, npoint, body, state0)
    row_base = lax.broadcasted_iota(jnp.int32, (B, npoint), 0) * n
    out_ref[...] = acc + row_base


def _mds_indices(x, y, z, s2):
    B, _ = x.shape
    npoint = x.shape[1] // 2
    return pl.pallas_call(
        _mds_body,
        out_shape=jax.ShapeDtypeStruct((B, npoint), jnp.int32),
    )(x, y, z, s2)


def _make_sc_gather(num_rows, row_words, num_idx):
    info = plsc.get_sparse_core_info()
    nw = info.num_cores * info.num_subcores
    chunk = 128
    assert num_idx % (nw * chunk) == 0
    per_w = num_idx // nw
    n_chunks = per_w // chunk
    mesh = plsc.VectorSubcoreMesh(core_axis_name="c", subcore_axis_name="s")

    @functools.partial(
        pl.kernel,
        mesh=mesh,
        out_type=jax.ShapeDtypeStruct((num_idx, row_words), jnp.float32),
        compiler_params=pltpu.CompilerParams(use_tc_tiling_on_sc=False),
        scratch_types=[
            pltpu.VMEM((n_chunks, chunk), jnp.int32),
            pltpu.VMEM((per_w, row_words), jnp.float32),
            pltpu.SemaphoreType.DMA,
        ],
    )
    def gather_k(table_hbm, idx_hbm, out_hbm, idx_v, rows_v, sem):
        wid = lax.axis_index("s") * info.num_cores + lax.axis_index("c")
        base = wid * per_w
        pltpu.sync_copy(idx_hbm.at[pl.ds(wid * n_chunks, n_chunks)], idx_v)
        for k in range(n_chunks):
            pltpu.async_copy(
                table_hbm.at[idx_v.at[k]],
                rows_v.at[pl.ds(k * chunk, chunk)],
                sem,
            )
        for k in range(n_chunks):
            pltpu.make_async_copy(
                table_hbm.at[idx_v.at[k]],
                rows_v.at[pl.ds(k * chunk, chunk)],
                sem,
            ).wait()
        pltpu.sync_copy(rows_v, out_hbm.at[pl.ds(base, per_w)])

    return gather_k


def kernel(partial, outs, out1_feat, mean_mst_dis):
    B, _, M = partial.shape
    N = outs.shape[2]
    C = out1_feat.shape[1]
    n = N + M
    F = 3 + C + 1
    Fp = (F + 7) // 8 * 8

    x = jnp.concatenate([outs[:, 0, :], partial[:, 0, :]], axis=1)
    y = jnp.concatenate([outs[:, 1, :], partial[:, 1, :]], axis=1)
    z = jnp.concatenate([outs[:, 2, :], partial[:, 2, :]], axis=1)
    s2 = (mean_mst_dis * mean_mst_dis)[:, None]

    gidx = _mds_indices(x, y, z, s2)

    top = jnp.concatenate(
        [
            jnp.transpose(outs, (0, 2, 1)),
            jnp.transpose(out1_feat, (0, 2, 1)),
            jnp.ones((B, N, 1), jnp.float32),
            jnp.zeros((B, N, Fp - F), jnp.float32),
        ],
        axis=2,
    )
    bot = jnp.concatenate(
        [
            jnp.transpose(partial, (0, 2, 1)),
            jnp.zeros((B, M, Fp - 3), jnp.float32),
        ],
        axis=2,
    )
    table = jnp.concatenate([top, bot], axis=1).reshape(B * n, Fp)

    rows = _make_sc_gather(B * n, Fp, B * N)(table, gidx.reshape(-1, 128))
    return jnp.transpose(rows.reshape(B, N, Fp), (0, 2, 1))[:, :F, :]


# speedup vs baseline: 23.6478x; 1.1535x over previous
import functools

import jax
import jax.numpy as jnp
from jax import lax
from jax.experimental import pallas as pl
from jax.experimental.pallas import tpu as pltpu
from jax.experimental.pallas import tpu_sc as plsc


def _mds_body(x_ref, y_ref, z_ref, s2_ref, out_ref, dens_ref):
    x = x_ref[...]
    y = y_ref[...]
    z = z_ref[...]
    s2 = s2_ref[...]
    B, n = x.shape
    npoint = out_ref.shape[1]
    lane_c = lax.broadcasted_iota(jnp.int32, (B, 128), 1)
    row_base = lax.broadcasted_iota(jnp.int32, (B, 1), 0) * n

    dens_ref[...] = jnp.zeros((B, n), x.dtype)

    buf0 = jnp.where(lane_c == 0, row_base, 0)

    nc = n // 128
    xc = [x[:, 128 * k:128 * (k + 1)] for k in range(nc)]
    yc = [y[:, 128 * k:128 * (k + 1)] for k in range(nc)]
    zc = [z[:, 128 * k:128 * (k + 1)] for k in range(nc)]

    def body(j, carry):
        px, py, pz, buf = carry
        dx = x - px
        dy = y - py
        dz = z - pz
        d2 = (dx * dx + dz * dz) + dy * dy
        density = dens_ref[...] + jnp.exp(-d2 / s2)
        dens_ref[...] = density

        dc = [density[:, 128 * k:128 * (k + 1)] for k in range(nc)]
        p = dc[0]
        for k in range(1, nc):
            p = jnp.minimum(p, dc[k])
        m = jnp.min(p, axis=1, keepdims=True)

        cand = [
            jnp.where(dc[k] == m, lane_c + 128 * k, n) for k in range(nc)
        ]
        cw = list(zip(cand, xc, yc, zc))
        gap = nc // 2
        while gap:
            nxt = []
            for i in range(gap):
                ca, xa, ya, za = cw[i]
                cb, xb, yb, zb = cw[i + gap]
                ta = ca <= cb
                nxt.append((
                    jnp.minimum(ca, cb),
                    jnp.where(ta, xa, xb),
                    jnp.where(ta, ya, yb),
                    jnp.where(ta, za, zb),
                ))
            cw = nxt
            gap //= 2
        c, wx, wy, wz = cw[0]

        ch = lax.convert_element_type(c, jnp.uint32) << 16
        xb_ = lax.bitcast_convert_type(wx, jnp.uint32)
        yb_ = lax.bitcast_convert_type(wy, jnp.uint32)
        zb_ = lax.bitcast_convert_type(wz, jnp.uint32)
        lo16 = jnp.uint32(0xFFFF)
        keys = [
            ch | (xb_ >> 16),
            ch | (xb_ & lo16),
            ch | (yb_ >> 16),
            ch | (yb_ & lo16),
            ch | (zb_ >> 16),
            ch | (zb_ & lo16),
        ]
        r = [
            lax.bitcast_convert_type(
                jnp.min(lax.bitcast_convert_type(kk, jnp.int32), axis=1, keepdims=True),
                jnp.uint32,
            )
            for kk in keys
        ]
        idxv = lax.convert_element_type(r[0] >> 16, jnp.int32)
        px = lax.bitcast_convert_type(((r[0] & lo16) << 16) | (r[1] & lo16), jnp.float32)
        py = lax.bitcast_convert_type(((r[2] & lo16) << 16) | (r[3] & lo16), jnp.float32)
        pz = lax.bitcast_convert_type(((r[4] & lo16) << 16) | (r[5] & lo16), jnp.float32)

        t = lax.rem(j, 128)
        buf = jnp.where(lane_c == t, idxv + row_base, buf)

        @pl.when(t == 127)
        def _flush():
            out_ref[:, pl.ds(pl.multiple_of(j - 127, 128), 128)] = buf

        return px, py, pz, buf

    lax.fori_loop(1, npoint, body, (x[:, 0:1], y[:, 0:1], z[:, 0:1], buf0))


def _mds_indices(x, y, z, s2):
    B, _ = x.shape
    npoint = x.shape[1] // 2
    return pl.pallas_call(
        _mds_body,
        out_shape=jax.ShapeDtypeStruct((B, npoint), jnp.int32),
        scratch_shapes=[pltpu.VMEM((B, x.shape[1]), jnp.float32)],
    )(x, y, z, s2)


def _make_sc_gather(num_rows, row_words, num_idx):
    info = plsc.get_sparse_core_info()
    nw = info.num_cores * info.num_subcores
    chunk = 128
    assert num_idx % (nw * chunk) == 0
    per_w = num_idx // nw
    n_chunks = per_w // chunk
    mesh = plsc.VectorSubcoreMesh(core_axis_name="c", subcore_axis_name="s")

    @functools.partial(
        pl.kernel,
        mesh=mesh,
        out_type=jax.ShapeDtypeStruct((num_idx, row_words), jnp.float32),
        compiler_params=pltpu.CompilerParams(use_tc_tiling_on_sc=False),
        scratch_types=[
            pltpu.VMEM((n_chunks, chunk), jnp.int32),
            pltpu.VMEM((per_w, row_words), jnp.float32),
            pltpu.SemaphoreType.DMA,
        ],
    )
    def gather_k(table_hbm, idx_hbm, out_hbm, idx_v, rows_v, sem):
        wid = lax.axis_index("s") * info.num_cores + lax.axis_index("c")
        base = wid * per_w
        pltpu.sync_copy(idx_hbm.at[pl.ds(wid * n_chunks, n_chunks)], idx_v)
        for k in range(n_chunks):
            pltpu.async_copy(
                table_hbm.at[idx_v.at[k]],
                rows_v.at[pl.ds(k * chunk, chunk)],
                sem,
            )
        for k in range(n_chunks):
            pltpu.make_async_copy(
                table_hbm.at[idx_v.at[k]],
                rows_v.at[pl.ds(k * chunk, chunk)],
                sem,
            ).wait()
        pltpu.sync_copy(rows_v, out_hbm.at[pl.ds(base, per_w)])

    return gather_k


def kernel(partial, outs, out1_feat, mean_mst_dis):
    B, _, M = partial.shape
    N = outs.shape[2]
    C = out1_feat.shape[1]
    n = N + M
    F = 3 + C + 1
    Fp = (F + 7) // 8 * 8

    x = jnp.concatenate([outs[:, 0, :], partial[:, 0, :]], axis=1)
    y = jnp.concatenate([outs[:, 1, :], partial[:, 1, :]], axis=1)
    z = jnp.concatenate([outs[:, 2, :], partial[:, 2, :]], axis=1)
    s2 = (mean_mst_dis * mean_mst_dis)[:, None]

    gidx = _mds_indices(x, y, z, s2)

    top = jnp.concatenate(
        [
            jnp.transpose(outs, (0, 2, 1)),
            jnp.transpose(out1_feat, (0, 2, 1)),
            jnp.ones((B, N, 1), jnp.float32),
            jnp.zeros((B, N, Fp - F), jnp.float32),
        ],
        axis=2,
    )
    bot = jnp.concatenate(
        [
            jnp.transpose(partial, (0, 2, 1)),
            jnp.zeros((B, M, Fp - 3), jnp.float32),
        ],
        axis=2,
    )
    table = jnp.concatenate([top, bot], axis=1).reshape(B * n, Fp)

    rows = _make_sc_gather(B * n, Fp, B * N)(table, gidx.reshape(-1, 128))
    return jnp.transpose(rows.reshape(B, N, Fp), (0, 2, 1))[:, :F, :]


# speedup vs baseline: 28.7656x; 1.2164x over previous
import functools

import jax
import jax.numpy as jnp
from jax import lax
from jax.experimental import pallas as pl
from jax.experimental.pallas import tpu as pltpu
from jax.experimental.pallas import tpu_sc as plsc


def _mds_body(x_ref, y_ref, z_ref, s2_ref, out_ref, dens_ref):
    x = x_ref[...]
    y = y_ref[...]
    z = z_ref[...]
    s2 = s2_ref[...]
    B, n = x.shape
    npoint = out_ref.shape[1]
    lane_c = lax.broadcasted_iota(jnp.int32, (B, 128), 1)
    row_base = lax.broadcasted_iota(jnp.int32, (B, 1), 0) * n

    dens_ref[...] = jnp.zeros((B, n), x.dtype)

    buf0 = jnp.where(lane_c == 0, row_base, 0)

    nc = n // 128
    xc = [x[:, 128 * k:128 * (k + 1)] for k in range(nc)]
    yc = [y[:, 128 * k:128 * (k + 1)] for k in range(nc)]
    zc = [z[:, 128 * k:128 * (k + 1)] for k in range(nc)]

    s2b = jnp.broadcast_to(s2, (B, 128))

    def body(j, carry):
        pxb, pyb, pzb, buf = carry
        dc = []
        for k in range(nc):
            dxk = xc[k] - pxb
            dyk = yc[k] - pyb
            dzk = zc[k] - pzb
            d2k = (dxk * dxk + dzk * dzk) + dyk * dyk
            dk = dens_ref[:, 128 * k:128 * (k + 1)] + jnp.exp(-d2k / s2b)
            dens_ref[:, 128 * k:128 * (k + 1)] = dk
            dc.append(dk)

        p = dc[0]
        for k in range(1, nc):
            p = jnp.minimum(p, dc[k])
        m = jnp.min(p, axis=1, keepdims=True)
        mb = jnp.broadcast_to(m, (B, 128))

        cand = [
            jnp.where(dc[k] == mb, lane_c + 128 * k, n) for k in range(nc)
        ]
        cw = list(zip(cand, xc, yc, zc))
        gap = nc // 2
        while gap:
            nxt = []
            for i in range(gap):
                ca, xa, ya, za = cw[i]
                cb, xb, yb, zb = cw[i + gap]
                ta = ca <= cb
                nxt.append((
                    jnp.minimum(ca, cb),
                    jnp.where(ta, xa, xb),
                    jnp.where(ta, ya, yb),
                    jnp.where(ta, za, zb),
                ))
            cw = nxt
            gap //= 2
        c, wx, wy, wz = cw[0]

        ch = lax.convert_element_type(c, jnp.uint32) << 16
        xb_ = lax.bitcast_convert_type(wx, jnp.uint32)
        yb_ = lax.bitcast_convert_type(wy, jnp.uint32)
        zb_ = lax.bitcast_convert_type(wz, jnp.uint32)
        lo16 = jnp.uint32(0xFFFF)
        keys = [
            ch | (xb_ >> 16),
            ch | (xb_ & lo16),
            ch | (yb_ >> 16),
            ch | (yb_ & lo16),
            ch | (zb_ >> 16),
            ch | (zb_ & lo16),
        ]
        r = [
            lax.bitcast_convert_type(
                jnp.min(lax.bitcast_convert_type(kk, jnp.int32), axis=1, keepdims=True),
                jnp.uint32,
            )
            for kk in keys
        ]
        idxv = lax.convert_element_type(r[0] >> 16, jnp.int32)
        px = lax.bitcast_convert_type(((r[0] & lo16) << 16) | (r[1] & lo16), jnp.float32)
        py = lax.bitcast_convert_type(((r[2] & lo16) << 16) | (r[3] & lo16), jnp.float32)
        pz = lax.bitcast_convert_type(((r[4] & lo16) << 16) | (r[5] & lo16), jnp.float32)
        pxb = jnp.broadcast_to(px, (B, 128))
        pyb = jnp.broadcast_to(py, (B, 128))
        pzb = jnp.broadcast_to(pz, (B, 128))

        t = lax.rem(j, 128)
        buf = jnp.where(lane_c == t, idxv + row_base, buf)

        @pl.when(t == 127)
        def _flush():
            out_ref[:, pl.ds(pl.multiple_of(j - 127, 128), 128)] = buf

        return pxb, pyb, pzb, buf

    lax.fori_loop(
        1,
        npoint,
        body,
        (
            jnp.broadcast_to(x[:, 0:1], (B, 128)),
            jnp.broadcast_to(y[:, 0:1], (B, 128)),
            jnp.broadcast_to(z[:, 0:1], (B, 128)),
            buf0,
        ),
    )


def _mds_indices(x, y, z, s2):
    B, _ = x.shape
    npoint = x.shape[1] // 2
    return pl.pallas_call(
        _mds_body,
        out_shape=jax.ShapeDtypeStruct((B, npoint), jnp.int32),
        scratch_shapes=[pltpu.VMEM((B, x.shape[1]), jnp.float32)],
    )(x, y, z, s2)


def _make_sc_gather(num_rows, row_words, num_idx):
    info = plsc.get_sparse_core_info()
    nw = info.num_cores * info.num_subcores
    chunk = 128
    assert num_idx % (nw * chunk) == 0
    per_w = num_idx // nw
    n_chunks = per_w // chunk
    mesh = plsc.VectorSubcoreMesh(core_axis_name="c", subcore_axis_name="s")

    @functools.partial(
        pl.kernel,
        mesh=mesh,
        out_type=jax.ShapeDtypeStruct((num_idx, row_words), jnp.float32),
        compiler_params=pltpu.CompilerParams(use_tc_tiling_on_sc=False),
        scratch_types=[
            pltpu.VMEM((n_chunks, chunk), jnp.int32),
            pltpu.VMEM((per_w, row_words), jnp.float32),
            pltpu.SemaphoreType.DMA,
        ],
    )
    def gather_k(table_hbm, idx_hbm, out_hbm, idx_v, rows_v, sem):
        wid = lax.axis_index("s") * info.num_cores + lax.axis_index("c")
        base = wid * per_w
        pltpu.sync_copy(idx_hbm.at[pl.ds(wid * n_chunks, n_chunks)], idx_v)
        for k in range(n_chunks):
            pltpu.async_copy(
                table_hbm.at[idx_v.at[k]],
                rows_v.at[pl.ds(k * chunk, chunk)],
                sem,
            )
        for k in range(n_chunks):
            pltpu.make_async_copy(
                table_hbm.at[idx_v.at[k]],
                rows_v.at[pl.ds(k * chunk, chunk)],
                sem,
            ).wait()
        pltpu.sync_copy(rows_v, out_hbm.at[pl.ds(base, per_w)])

    return gather_k


def kernel(partial, outs, out1_feat, mean_mst_dis):
    B, _, M = partial.shape
    N = outs.shape[2]
    C = out1_feat.shape[1]
    n = N + M
    F = 3 + C + 1
    Fp = (F + 7) // 8 * 8

    x = jnp.concatenate([outs[:, 0, :], partial[:, 0, :]], axis=1)
    y = jnp.concatenate([outs[:, 1, :], partial[:, 1, :]], axis=1)
    z = jnp.concatenate([outs[:, 2, :], partial[:, 2, :]], axis=1)
    s2 = (mean_mst_dis * mean_mst_dis)[:, None]

    gidx = _mds_indices(x, y, z, s2)

    top = jnp.concatenate(
        [
            jnp.transpose(outs, (0, 2, 1)),
            jnp.transpose(out1_feat, (0, 2, 1)),
            jnp.ones((B, N, 1), jnp.float32),
            jnp.zeros((B, N, Fp - F), jnp.float32),
        ],
        axis=2,
    )
    bot = jnp.concatenate(
        [
            jnp.transpose(partial, (0, 2, 1)),
            jnp.zeros((B, M, Fp - 3), jnp.float32),
        ],
        axis=2,
    )
    table = jnp.concatenate([top, bot], axis=1).reshape(B * n, Fp)

    rows = _make_sc_gather(B * n, Fp, B * N)(table, gidx.reshape(-1, 128))
    return jnp.transpose(rows.reshape(B, N, Fp), (0, 2, 1))[:, :F, :]
